# Initial kernel scaffold; baseline (speedup 1.0000x reference)
import jax
import jax.numpy as jnp
from jax.experimental import pallas as pl


def kernel(x, W, b, dynamic_bias):
    raise NotImplementedError("write your pallas kernel here")


# speedup vs baseline: 5.2491x; 5.2491x over previous
import functools

import jax
import jax.numpy as jnp
from jax import lax
from jax.experimental import pallas as pl
from jax.experimental.pallas import tpu as pltpu

K_TOP = 8
TM = 512


def _gate_block(x_ref, w_ref, bias_ref, o_ref):
    acc = lax.dot_general(
        x_ref[...], w_ref[...],
        dimension_numbers=(((1,), (1,)), ((), ())),
        preferred_element_type=jnp.float32,
    )
    logits = acc + bias_ref[...]
    tm, e = logits.shape
    iota = lax.broadcasted_iota(jnp.int32, (tm, e), 1)
    work = logits
    selected = jnp.zeros((tm, e), dtype=jnp.bool_)
    row_max = jnp.max(work, axis=1, keepdims=True)
    for i in range(K_TOP):
        m = row_max if i == 0 else jnp.max(work, axis=1, keepdims=True)
        cand = jnp.where(work == m, iota, e)
        first = jnp.min(cand, axis=1, keepdims=True)
        sel = iota == first
        selected = jnp.logical_or(selected, sel)
        work = jnp.where(sel, -jnp.inf, work)
    p = jnp.where(selected, jnp.exp(logits - row_max), 0.0)
    denom = jnp.sum(p, axis=1, keepdims=True)
    o_ref[...] = p / denom


@jax.jit
def _gate(x2d, W, bias2d):
    n, d = x2d.shape
    e = W.shape[0]
    grid = n // TM
    return pl.pallas_call(
        _gate_block,
        grid=(grid,),
        in_specs=[
            pl.BlockSpec((TM, d), lambda i: (i, 0)),
            pl.BlockSpec((e, d), lambda i: (0, 0)),
            pl.BlockSpec((1, e), lambda i: (0, 0)),
        ],
        out_specs=pl.BlockSpec((TM, e), lambda i: (i, 0)),
        out_shape=jax.ShapeDtypeStruct((n, e), jnp.float32),
        compiler_params=pltpu.CompilerParams(
            dimension_semantics=("arbitrary",),
        ),
    )(x2d, W, bias2d)


def kernel(x, W, b, dynamic_bias):
    B, T, D = x.shape
    E = W.shape[0]
    x2d = x.reshape(B * T, D)
    bias2d = (b + dynamic_bias).reshape(1, E)
    out = _gate(x2d, W, bias2d)
    return out.reshape(B, T, E)


# speedup vs baseline: 6.2055x; 1.1822x over previous
import functools

import jax
import jax.numpy as jnp
from jax import lax
from jax.experimental import pallas as pl
from jax.experimental.pallas import tpu as pltpu

K_TOP = 8
TM = 512


def _gate_block(x_ref, w_ref, bias_ref, o_ref):
    acc = lax.dot_general(
        x_ref[...], w_ref[...],
        dimension_numbers=(((1,), (1,)), ((), ())),
        preferred_element_type=jnp.float32,
    )
    logits = acc + bias_ref[...]
    tm, e = logits.shape
    work = logits
    row_max = jnp.max(work, axis=1, keepdims=True)
    m = row_max
    for _ in range(K_TOP - 1):
        work = jnp.where(work == m, -jnp.inf, work)
        m = jnp.max(work, axis=1, keepdims=True)
    t = m
    gtf = (logits > t).astype(jnp.float32)
    eqf = (logits == t).astype(jnp.float32)
    r_i = lax.broadcasted_iota(jnp.int32, (e, e), 0)
    c_i = lax.broadcasted_iota(jnp.int32, (e, e), 1)
    strict_lt = (r_i < c_i).astype(jnp.float32)
    ones_ee = jnp.ones((e, e), dtype=jnp.float32)
    n_gt = lax.dot_general(gtf, ones_ee, (((1,), (0,)), ((), ())),
                           preferred_element_type=jnp.float32)
    rank = lax.dot_general(eqf, strict_lt, (((1,), (0,)), ((), ())),
                           preferred_element_type=jnp.float32)
    keep = jnp.float32(K_TOP) - n_gt
    sel = jnp.logical_or(logits > t,
                         jnp.logical_and(logits == t, rank < keep))
    p = jnp.where(sel, jnp.exp(logits - row_max), 0.0)
    denom = lax.dot_general(p, ones_ee, (((1,), (0,)), ((), ())),
                            preferred_element_type=jnp.float32)
    o_ref[...] = p / denom


@jax.jit
def _gate(x2d, W, bias2d):
    n, d = x2d.shape
    e = W.shape[0]
    grid = n // TM
    return pl.pallas_call(
        _gate_block,
        grid=(grid,),
        in_specs=[
            pl.BlockSpec((TM, d), lambda i: (i, 0)),
            pl.BlockSpec((e, d), lambda i: (0, 0)),
            pl.BlockSpec((1, e), lambda i: (0, 0)),
        ],
        out_specs=pl.BlockSpec((TM, e), lambda i: (i, 0)),
        out_shape=jax.ShapeDtypeStruct((n, e), jnp.float32),
        compiler_params=pltpu.CompilerParams(
            dimension_semantics=("arbitrary",),
        ),
    )(x2d, W, bias2d)


def kernel(x, W, b, dynamic_bias):
    B, T, D = x.shape
    E = W.shape[0]
    x2d = x.reshape(B * T, D)
    bias2d = (b + dynamic_bias).reshape(1, E)
    out = _gate(x2d, W, bias2d)
    return out.reshape(B, T, E)


# speedup vs baseline: 6.8956x; 1.1112x over previous
import functools

import jax
import jax.numpy as jnp
from jax import lax
from jax.experimental import pallas as pl
from jax.experimental.pallas import tpu as pltpu

K_TOP = 8
TM = 1024


def _gate_block(x_ref, w_ref, bias_ref, o_ref):
    acc = lax.dot_general(
        x_ref[...], w_ref[...],
        dimension_numbers=(((1,), (1,)), ((), ())),
        preferred_element_type=jnp.float32,
    )
    logits = acc + bias_ref[...]
    tm, e = logits.shape
    work = logits
    row_max = jnp.max(work, axis=1, keepdims=True)
    m = row_max
    for _ in range(K_TOP - 1):
        work = jnp.where(work == m, -jnp.inf, work)
        m = jnp.max(work, axis=1, keepdims=True)
    t = m
    gtf = (logits > t).astype(jnp.float32)
    eqf = (logits == t).astype(jnp.float32)
    r_i = lax.broadcasted_iota(jnp.int32, (e, e), 0)
    c_i = lax.broadcasted_iota(jnp.int32, (e, e), 1)
    strict_lt = (r_i < c_i).astype(jnp.float32)
    ones_ee = jnp.ones((e, e), dtype=jnp.float32)
    n_gt = lax.dot_general(gtf, ones_ee, (((1,), (0,)), ((), ())),
                           preferred_element_type=jnp.float32)
    rank = lax.dot_general(eqf, strict_lt, (((1,), (0,)), ((), ())),
                           preferred_element_type=jnp.float32)
    keep = jnp.float32(K_TOP) - n_gt
    sel = jnp.logical_or(logits > t,
                         jnp.logical_and(logits == t, rank < keep))
    p = jnp.where(sel, jnp.exp(logits - row_max), 0.0)
    denom = lax.dot_general(p, ones_ee, (((1,), (0,)), ((), ())),
                            preferred_element_type=jnp.float32)
    o_ref[...] = p / denom


@jax.jit
def _gate(x2d, W, bias2d):
    n, d = x2d.shape
    e = W.shape[0]
    grid = n // TM
    return pl.pallas_call(
        _gate_block,
        grid=(grid,),
        in_specs=[
            pl.BlockSpec((TM, d), lambda i: (i, 0)),
            pl.BlockSpec((e, d), lambda i: (0, 0)),
            pl.BlockSpec((1, e), lambda i: (0, 0)),
        ],
        out_specs=pl.BlockSpec((TM, e), lambda i: (i, 0)),
        out_shape=jax.ShapeDtypeStruct((n, e), jnp.float32),
        compiler_params=pltpu.CompilerParams(
            dimension_semantics=("arbitrary",),
        ),
    )(x2d, W, bias2d)


def kernel(x, W, b, dynamic_bias):
    B, T, D = x.shape
    E = W.shape[0]
    x2d = x.reshape(B * T, D)
    bias2d = (b + dynamic_bias).reshape(1, E)
    out = _gate(x2d, W, bias2d)
    return out.reshape(B, T, E)
